# baseline (speedup 1.0000x reference)
import functools

import jax
import jax.numpy as jnp
from jax import lax
from jax.experimental import pallas as pl
from jax.experimental.pallas import tpu as pltpu
from jax.experimental.pallas import tpu_sc as plsc

DIM = 64
HDIM = DIM // 2
B = 16384
ROWS = 1000000
NC = 2
NS = 16
NW = NC * NS
BPW = B // NW
CHUNK = 64
NCHUNK = BPW // CHUNK
L = 16
PACK = 4 * HDIM


def _lane_shuffle(v, perm):
    return lax.gather(
        v, perm[:, None],
        dimension_numbers=lax.GatherDimensionNumbers(
            offset_dims=(), collapsed_slice_dims=(0,),
            start_index_map=(0,)),
        slice_sizes=(1,),
        mode=lax.GatherScatterMode.PROMISE_IN_BOUNDS)


def _make_sc_call():
    mesh = plsc.VectorSubcoreMesh(core_axis_name="c", subcore_axis_name="s")

    @functools.partial(
        pl.kernel,
        mesh=mesh,
        out_type=jax.ShapeDtypeStruct((B,), jnp.float32),
        compiler_params=pltpu.CompilerParams(
            needs_layout_passes=False, use_tc_tiling_on_sc=True),
        scratch_types=[
            pltpu.VMEM((CHUNK,), jnp.int32),
            pltpu.VMEM((CHUNK,), jnp.int32),
            pltpu.VMEM((CHUNK,), jnp.int32),
            pltpu.VMEM((CHUNK,), jnp.int32),
            pltpu.VMEM((CHUNK, PACK), jnp.float32),
            pltpu.VMEM((CHUNK, PACK), jnp.float32),
            pltpu.VMEM((CHUNK, PACK), jnp.float32),
            pltpu.VMEM((CHUNK, PACK), jnp.float32),
            pltpu.VMEM((CHUNK, PACK), jnp.float32),
            pltpu.VMEM((CHUNK, PACK), jnp.float32),
            pltpu.VMEM((CHUNK, PACK), jnp.float32),
            pltpu.VMEM((CHUNK, PACK), jnp.float32),
            pltpu.VMEM((CHUNK,), jnp.float32),
            pltpu.SemaphoreType.DMA,
            pltpu.SemaphoreType.DMA,
            pltpu.SemaphoreType.DMA,
            pltpu.SemaphoreType.DMA,
            pltpu.SemaphoreType.DMA,
            pltpu.SemaphoreType.DMA,
            pltpu.SemaphoreType.DMA,
            pltpu.SemaphoreType.DMA,
        ],
    )
    def sc_call(idx_a_hbm, idx_b_hbm,
                minlo_hbm, minhi_hbm, maxlo_hbm, maxhi_hbm, out_hbm,
                ia_v, ib_v, ha_v, hb_v,
                mla_v, mha_v, xla_v, xha_v,
                mlb_v, mhb_v, xlb_v, xhb_v, out_v,
                s0, s1, s2, s3, s4, s5, s6, s7):
        wid = lax.axis_index("s") * NC + lax.axis_index("c")
        lanes = lax.iota(jnp.int32, L)
        perms = [lanes ^ k for k in (1, 2, 4, 8)]
        for chunk in range(NCHUNK):
            base = wid * BPW + chunk * CHUNK
            pltpu.sync_copy(idx_a_hbm.at[pl.ds(base, CHUNK)], ia_v)
            pltpu.sync_copy(idx_b_hbm.at[pl.ds(base, CHUNK)], ib_v)
            for g in range(CHUNK // L):
                sl = pl.ds(g * L, L)
                ha_v[sl] = jax.lax.shift_right_logical(ia_v[sl], 2)
                hb_v[sl] = jax.lax.shift_right_logical(ib_v[sl], 2)
            cps = [
                pltpu.async_copy(minlo_hbm.at[ha_v], mla_v, s0),
                pltpu.async_copy(minhi_hbm.at[ha_v], mha_v, s1),
                pltpu.async_copy(maxlo_hbm.at[ha_v], xla_v, s2),
                pltpu.async_copy(maxhi_hbm.at[ha_v], xha_v, s3),
                pltpu.async_copy(minlo_hbm.at[hb_v], mlb_v, s4),
                pltpu.async_copy(minhi_hbm.at[hb_v], mhb_v, s5),
                pltpu.async_copy(maxlo_hbm.at[hb_v], xlb_v, s6),
                pltpu.async_copy(maxhi_hbm.at[hb_v], xhb_v, s7),
            ]
            for c in cps:
                c.wait()

            def pair_step(p, acc):
                lane_p = lanes * 0 + (p % L)
                grp = pl.ds((p // L) * L, L)
                qa = _lane_shuffle(ia_v[grp], lane_p) & 3
                qb = _lane_shuffle(ib_v[grp], lane_p) & 3

                def pick(ref, q):
                    outs = []
                    for g in range(2):
                        x0 = ref[p, pl.ds(0 * HDIM + g * L, L)]
                        x1 = ref[p, pl.ds(1 * HDIM + g * L, L)]
                        x2 = ref[p, pl.ds(2 * HDIM + g * L, L)]
                        x3 = ref[p, pl.ds(3 * HDIM + g * L, L)]
                        v = jnp.where(q == 3, x3,
                                      jnp.where(q == 2, x2,
                                                jnp.where(q == 1, x1, x0)))
                        outs.append(v)
                    return outs

                mina = pick(mla_v, qa) + pick(mha_v, qa)
                maxa = pick(xla_v, qa) + pick(xha_v, qa)
                minb = pick(mlb_v, qb) + pick(mhb_v, qb)
                maxb = pick(xlb_v, qb) + pick(xhb_v, qb)
                ratio = jnp.full((L,), 1.0, jnp.float32)
                for g in range(4):
                    wa = maxa[g] - mina[g]
                    wi = (jnp.minimum(maxa[g], maxb[g])
                          - jnp.maximum(mina[g], minb[g]))
                    ratio = ratio * (wi / wa)
                for perm in perms:
                    ratio = ratio * _lane_shuffle(ratio, perm)
                acc = jnp.where(lanes == (p % L), ratio, acc)

                @pl.when((p % L) == (L - 1))
                def _():
                    out_v[pl.ds((p // L) * L, L)] = acc
                return acc

            lax.fori_loop(0, CHUNK, pair_step,
                          jnp.full((L,), 0.0, jnp.float32))
            pltpu.sync_copy(out_v, out_hbm.at[pl.ds(base, CHUNK)])

    return sc_call


_SC_CALL = _make_sc_call()


def kernel(x, min_embeddings, max_embeddings, relation_embeddings):
    idx_a = x[:, 0]
    idx_b = x[:, 1]
    min_lo = min_embeddings[:, :HDIM].reshape(ROWS // 4, PACK)
    min_hi = min_embeddings[:, HDIM:].reshape(ROWS // 4, PACK)
    max_lo = max_embeddings[:, :HDIM].reshape(ROWS // 4, PACK)
    max_hi = max_embeddings[:, HDIM:].reshape(ROWS // 4, PACK)
    out = _SC_CALL(idx_a, idx_b, min_lo, min_hi, max_lo, max_hi)
    return out.reshape(B, 1)
